# baseline (speedup 1.0000x reference)
import jax
import jax.numpy as jnp
from jax import lax
from jax.experimental import pallas as pl
from jax.experimental.pallas import tpu as pltpu
from jax.experimental.pallas import tpu_sc as plsc

NUM_CLASSES = 1000
EMB_DIM = 64
BATCH = 16384

_INFO = plsc.get_sparse_core_info()
NS = _INFO.num_subcores
CHUNK = 512
B_PER_W = BATCH // NS
NCH = B_PER_W // CHUNK


def _gather_body(idx_hbm, table_hbm, out_hbm, idx_v, rows_v, *sems):
    gsems, wsem = sems[:NCH], sems[NCH]
    wid = lax.axis_index("s")
    pltpu.sync_copy(idx_hbm.at[wid], idx_v)
    gathers = [
        pltpu.async_copy(table_hbm.at[idx_v.at[j]], rows_v.at[j], gsems[j])
        for j in range(NCH)
    ]
    writes = []
    for j in range(NCH):
        gathers[j].wait()
        writes.append(pltpu.async_copy(rows_v.at[j], out_hbm.at[wid, j], wsem))
    for cp in writes:
        cp.wait()


_gather = pl.kernel(
    _gather_body,
    out_type=jax.ShapeDtypeStruct((NS, NCH, CHUNK, EMB_DIM), jnp.float32),
    mesh=plsc.VectorSubcoreMesh(
        core_axis_name="c", subcore_axis_name="s", num_cores=1
    ),
    scratch_types=[
        pltpu.VMEM((NCH, CHUNK), jnp.int32),
        pltpu.VMEM((NCH, CHUNK, EMB_DIM), jnp.float32),
    ] + [pltpu.SemaphoreType.DMA] * (NCH + 1),
    compiler_params=pltpu.CompilerParams(use_tc_tiling_on_sc=False),
)


def kernel(labels, emb_table):
    idx = labels.astype(jnp.int32).reshape(NS, NCH, CHUNK)
    out = _gather(idx, emb_table)
    return out.reshape(BATCH, EMB_DIM)


# speedup vs baseline: 1.0019x; 1.0019x over previous
import jax
import jax.numpy as jnp
from jax import lax
from jax.experimental import pallas as pl
from jax.experimental.pallas import tpu as pltpu
from jax.experimental.pallas import tpu_sc as plsc

NUM_CLASSES = 1000
EMB_DIM = 64
BATCH = 16384

_INFO = plsc.get_sparse_core_info()
NS = _INFO.num_subcores
CHUNK = 256
B_PER_W = BATCH // NS
NCH = B_PER_W // CHUNK


def _gather_body(idx_hbm, table_hbm, out_hbm, idx_v, rows_v, sem):
    wid = lax.axis_index("s")
    pltpu.sync_copy(idx_hbm.at[wid], idx_v)
    copies = [
        pltpu.async_copy(table_hbm.at[idx_v.at[j]], rows_v.at[j], sem)
        for j in range(NCH)
    ]
    for cp in copies:
        cp.wait()
    pltpu.sync_copy(rows_v, out_hbm.at[wid])


_gather = pl.kernel(
    _gather_body,
    out_type=jax.ShapeDtypeStruct((NS, NCH, CHUNK, EMB_DIM), jnp.float32),
    mesh=plsc.VectorSubcoreMesh(
        core_axis_name="c", subcore_axis_name="s", num_cores=1
    ),
    scratch_types=[
        pltpu.VMEM((NCH, CHUNK), jnp.int32),
        pltpu.VMEM((NCH, CHUNK, EMB_DIM), jnp.float32),
        pltpu.SemaphoreType.DMA,
    ],
    compiler_params=pltpu.CompilerParams(use_tc_tiling_on_sc=False),
)


def kernel(labels, emb_table):
    idx = labels.astype(jnp.int32).reshape(NS, NCH, CHUNK)
    out = _gather(idx, emb_table)
    return out.reshape(BATCH, EMB_DIM)
